# Initial kernel scaffold; baseline (speedup 1.0000x reference)
import jax
import jax.numpy as jnp
from jax.experimental import pallas as pl


def kernel(proxy_points, centers, logits):
    raise NotImplementedError("write your pallas kernel here")


# speedup vs baseline: 56796.8518x; 56796.8518x over previous
import jax
import jax.numpy as jnp
from jax.experimental import pallas as pl
from jax.experimental.pallas import tpu as pltpu

NUM_CLIENTS = 1024
NUM_CLUSTERS = 64
PCA_DIM = 4
SINKHORN_REG = 0.2
TEMPERATURE = 0.5


def _vwc_body(centers_ref, ct_ref, logits_ref, probs_ref, loss_ref):
    c = centers_ref[...]
    ct = ct_ref[...]
    lg = logits_ref[...]

    cn2_row = jnp.sum(ct * ct, axis=0, keepdims=True)
    cn_row = jnp.where(cn2_row > 0,
                       jnp.sqrt(jnp.where(cn2_row > 0, cn2_row, 1.0)), 0.0)

    a = lg - (1.0 / TEMPERATURE) * cn_row
    m = jnp.max(a, axis=1, keepdims=True)
    e = jnp.exp(a - m)
    s = jnp.sum(e, axis=1, keepdims=True)
    probs = e / s
    probs_ref[...] = probs

    colsum = jnp.sum(probs, axis=0, keepdims=True)
    cluster_probs = colsum * (1.0 / NUM_CLIENTS)
    entropy = -jnp.sum(cluster_probs * jnp.log(cluster_probs + 1e-10))

    cn2_col = jnp.sum(c * c, axis=1, keepdims=True)
    gram = jnp.dot(c, ct, preferred_element_type=jnp.float32)
    pd2 = cn2_col + cn2_row - 2.0 * gram
    pd = jnp.where(pd2 > 0, jnp.sqrt(jnp.where(pd2 > 0, pd2, 1.0)), 0.0)
    iota_r = jax.lax.broadcasted_iota(jnp.int32, (NUM_CLUSTERS, NUM_CLUSTERS), 0)
    iota_c = jax.lax.broadcasted_iota(jnp.int32, (NUM_CLUSTERS, NUM_CLUSTERS), 1)
    pd = pd + jnp.where(iota_r == iota_c, 1e10, 0.0)
    min_dist = -jnp.min(pd)

    rowmax = jnp.max(probs, axis=1, keepdims=True)
    kio = jax.lax.broadcasted_iota(jnp.int32, (NUM_CLIENTS, NUM_CLUSTERS), 1)
    idx = jnp.min(jnp.where(probs == rowmax, kio, NUM_CLUSTERS),
                  axis=1, keepdims=True)
    onehot = (idx == kio).astype(jnp.float32)
    counts = jnp.sum(onehot, axis=0, keepdims=True)

    total = jnp.sum(counts)
    proportions = counts / total
    gini = jnp.sum(proportions * (1.0 - proportions))
    mean_count = total * (1.0 / NUM_CLUSTERS)
    std_count = jnp.sqrt(jnp.mean((counts - mean_count) ** 2))
    imbalance = std_count / (mean_count + 1e-10)

    distance_loss = jnp.sum(colsum * cn_row)
    loss = (distance_loss - SINKHORN_REG * entropy + 0.2 * min_dist
            + 0.5 * gini + 0.8 * imbalance)
    loss_ref[0, 0] = loss


def kernel(proxy_points, centers, logits):
    del proxy_points
    probs, loss = pl.pallas_call(
        _vwc_body,
        out_shape=(
            jax.ShapeDtypeStruct((NUM_CLIENTS, NUM_CLUSTERS), jnp.float32),
            jax.ShapeDtypeStruct((1, 1), jnp.float32),
        ),
        out_specs=(
            pl.BlockSpec(memory_space=pltpu.VMEM),
            pl.BlockSpec(memory_space=pltpu.SMEM),
        ),
        in_specs=(
            pl.BlockSpec(memory_space=pltpu.VMEM),
            pl.BlockSpec(memory_space=pltpu.VMEM),
            pl.BlockSpec(memory_space=pltpu.VMEM),
        ),
    )(centers, centers.T, logits)
    return probs, loss.reshape(())
